# Initial kernel scaffold; baseline (speedup 1.0000x reference)
import functools

import jax
import jax.numpy as jnp
from jax import lax
from jax.experimental import pallas as pl
from jax.experimental.pallas import tpu as pltpu
from jax.experimental.pallas import tpu_sc as plsc

N = 10000
E = 320000
F = 128

NC, NS = 2, 16
NW = NC * NS
EPT = E // NW
CH = 80
NCHUNK = EPT // CH
RPT = N // NS
ZR = 125

_mesh = plsc.VectorSubcoreMesh(
    core_axis_name="c", subcore_axis_name="s", num_cores=NC, num_subcores=NS
)


def _make_sc_segsum(with_deg: bool):
  out_type = [jax.ShapeDtypeStruct((NC, N, F), jnp.float32)]
  scratch = [
      pltpu.VMEM((CH,), jnp.int32),
      pltpu.VMEM((CH,), jnp.int32),
      pltpu.VMEM((CH, F), jnp.float32),
      pltpu.VMEM((ZR, F), jnp.float32),
      pltpu.VMEM_SHARED((N, F), jnp.float32),
      pltpu.SemaphoreType.DMA,
  ]
  if with_deg:
    out_type.append(jax.ShapeDtypeStruct((NC, N, 16), jnp.float32))
    scratch += [
        pltpu.VMEM((CH, 16), jnp.float32),
        pltpu.VMEM((ZR, 16), jnp.float32),
        pltpu.VMEM_SHARED((N, 16), jnp.float32),
    ]

  def body(y_hbm, src_hbm, dst_hbm, agg_out, *rest):
    if with_deg:
      (deg_out, sidx, didx, rows, zbuf, agg_sp, sem,
       ones16, zdeg, deg_sp) = rest
    else:
      sidx, didx, rows, zbuf, agg_sp, sem = rest
    c = lax.axis_index("c")
    s = lax.axis_index("s")
    tid = c * NS + s
    zv = jnp.zeros((16,), jnp.float32)

    def zfill(r, carry):
      for cc in range(F // 16):
        zbuf[r, pl.ds(cc * 16, 16)] = zv
      if with_deg:
        zdeg[r, :] = zv
      return carry

    lax.fori_loop(0, ZR, zfill, 0)
    if with_deg:
      ov = jnp.full((16,), 1.0, jnp.float32)

      def ofill(r, carry):
        ones16[r, :] = ov
        return carry

      lax.fori_loop(0, CH, ofill, 0)

    row0 = s * RPT
    for b in range(RPT // ZR):
      pltpu.sync_copy(zbuf, agg_sp.at[pl.ds(row0 + b * ZR, ZR)])
      if with_deg:
        pltpu.sync_copy(zdeg, deg_sp.at[pl.ds(row0 + b * ZR, ZR)])
    plsc.subcore_barrier()

    ebase = tid * EPT

    def step(k, carry):
      off = pl.multiple_of(ebase + k * CH, 8)
      pltpu.sync_copy(src_hbm.at[pl.ds(off, CH)], sidx)
      pltpu.sync_copy(dst_hbm.at[pl.ds(off, CH)], didx)
      pltpu.async_copy(y_hbm.at[sidx], rows, sem).wait()
      pltpu.sync_copy(rows, agg_sp.at[didx], add=True)
      if with_deg:
        pltpu.sync_copy(ones16, deg_sp.at[didx], add=True)
      return carry

    lax.fori_loop(0, NCHUNK, step, 0)
    plsc.subcore_barrier()

    pltpu.sync_copy(agg_sp.at[pl.ds(row0, RPT)],
                    agg_out.at[c].at[pl.ds(row0, RPT)])
    if with_deg:
      pltpu.sync_copy(deg_sp.at[pl.ds(row0, RPT)],
                      deg_out.at[c].at[pl.ds(row0, RPT)])

  return pl.kernel(body, out_type=out_type, mesh=_mesh, scratch_types=scratch,
                   name="sc_segsum_deg" if with_deg else "sc_segsum")


_sc_segsum_deg = _make_sc_segsum(True)
_sc_segsum = _make_sc_segsum(False)

BN = 400


def _ln_relu(h, g, b):
  m = jnp.mean(h, axis=-1, keepdims=True)
  v = jnp.mean((h - m) * (h - m), axis=-1, keepdims=True)
  h = (h - m) * lax.rsqrt(v + 1e-5) * g + b
  return jnp.maximum(h, 0.0)


def _tc1_body(aggp, degp, x, wlT, bl, wrT, g, be, h_out):
  deg = degp[0, :, 0:1] + degp[1, :, 0:1]
  agg = (aggp[0] + aggp[1]) / jnp.maximum(deg, 1.0)
  h = (jnp.dot(agg, wlT[...], preferred_element_type=jnp.float32) + bl[...]
       + jnp.dot(x[...], wrT[...], preferred_element_type=jnp.float32))
  h_out[...] = _ln_relu(h, g[...], be[...])


def _tc2_body(aggp, degp, x, wlT, bl, wrT, g, be, wpT, bp, out, acc):
  i = pl.program_id(0)

  @pl.when(i == 0)
  def _():
    acc[...] = jnp.zeros_like(acc)

  deg = degp[0, :, 0:1] + degp[1, :, 0:1]
  agg = (aggp[0] + aggp[1]) / jnp.maximum(deg, 1.0)
  h = (jnp.dot(agg, wlT[...], preferred_element_type=jnp.float32) + bl[...]
       + jnp.dot(x[...], wrT[...], preferred_element_type=jnp.float32))
  h = _ln_relu(h, g[...], be[...])
  acc[...] += jnp.sum(h, axis=0, keepdims=True)

  @pl.when(i == pl.num_programs(0) - 1)
  def _():
    pooled = acc[...] * (1.0 / N)
    out[...] = (jnp.dot(pooled, wpT[...], preferred_element_type=jnp.float32)
                + bp[...])


_row_spec = pl.BlockSpec((BN, F), lambda i: (i, 0))
_aggp_spec = pl.BlockSpec((NC, BN, F), lambda i: (0, i, 0))
_degp_spec = pl.BlockSpec((NC, BN, 16), lambda i: (0, i, 0))
_w_spec = pl.BlockSpec((F, F), lambda i: (0, 0))
_v_spec = pl.BlockSpec((1, F), lambda i: (0, 0))

_tc1 = pl.pallas_call(
    _tc1_body,
    grid=(N // BN,),
    in_specs=[_aggp_spec, _degp_spec, _row_spec,
              _w_spec, _v_spec, _w_spec, _v_spec, _v_spec],
    out_specs=_row_spec,
    out_shape=jax.ShapeDtypeStruct((N, F), jnp.float32),
)

_tc2 = pl.pallas_call(
    _tc2_body,
    grid=(N // BN,),
    in_specs=[_aggp_spec, _degp_spec, _row_spec,
              _w_spec, _v_spec, _w_spec, _v_spec, _v_spec,
              _w_spec, _v_spec],
    out_specs=pl.BlockSpec((1, F), lambda i: (0, 0)),
    out_shape=jax.ShapeDtypeStruct((1, F), jnp.float32),
    scratch_shapes=[pltpu.VMEM((1, F), jnp.float32)],
)


def kernel(x, edge_index, batch, W_l1, b_l1, W_r1, g1, be1,
           W_l2, b_l2, W_r2, g2, be2, W_p, b_p):
  src = edge_index[0]
  dst = edge_index[1]
  agg1p, degp = _sc_segsum_deg(x, src, dst)
  h1 = _tc1(agg1p, degp, x,
            W_l1.T, b_l1.reshape(1, F), W_r1.T, g1.reshape(1, F),
            be1.reshape(1, F))
  (agg2p,) = _sc_segsum(h1, src, dst)
  out = _tc2(agg2p, degp, h1,
             W_l2.T, b_l2.reshape(1, F), W_r2.T, g2.reshape(1, F),
             be2.reshape(1, F), W_p.T, b_p.reshape(1, F))
  return out.reshape(F)


# speedup vs baseline: 4.6963x; 4.6963x over previous
import functools

import jax
import jax.numpy as jnp
from jax import lax
from jax.experimental import pallas as pl
from jax.experimental.pallas import tpu as pltpu
from jax.experimental.pallas import tpu_sc as plsc

N = 10000
E = 320000
F = 128

NC, NS = 2, 16
NW = NC * NS
EPT = E // NW
CH = 80
NCHUNK = EPT // CH
RA = 624
SR = 48
NCOPY = RA // SR
TAIL0 = NS * RA
TAIL = N - TAIL0

_mesh = plsc.VectorSubcoreMesh(
    core_axis_name="c", subcore_axis_name="s", num_cores=NC, num_subcores=NS
)


def _zero_shared(s, zbuf, sp):
  zv = jnp.zeros((16,), jnp.float32)

  def zfill(r, carry):
    for cc in range(F // 16):
      zbuf[r, pl.ds(cc * 16, 16)] = zv
    return carry

  lax.fori_loop(0, SR, zfill, 0)
  row0 = pl.multiple_of(s * RA, 8)
  for b in range(NCOPY):
    pltpu.sync_copy(zbuf, sp.at[pl.ds(row0 + b * SR, SR)])

  @pl.when(s == NS - 1)
  def _():
    pltpu.sync_copy(zbuf.at[pl.ds(0, TAIL)], sp.at[pl.ds(TAIL0, TAIL)])

  return row0


def _copy_out(c, s, row0, sp, out):
  for b in range(NCOPY):
    r = row0 + b * SR
    pltpu.sync_copy(sp.at[pl.ds(r, SR)], out.at[c].at[pl.ds(r, SR)])

  @pl.when(s == NS - 1)
  def _():
    pltpu.sync_copy(sp.at[pl.ds(TAIL0, TAIL)], out.at[c].at[pl.ds(TAIL0, TAIL)])


def _segsum_body(y_hbm, src_hbm, dst_hbm, agg_out,
                 sidx, didx, rows, zbuf, agg_sp, sem):
  c = lax.axis_index("c")
  s = lax.axis_index("s")
  tid = c * NS + s
  row0 = _zero_shared(s, zbuf, agg_sp)
  plsc.subcore_barrier()

  ebase = tid * EPT

  def step(k, carry):
    off = pl.multiple_of(ebase + k * CH, 8)
    pltpu.sync_copy(src_hbm.at[pl.ds(off, CH)], sidx)
    pltpu.sync_copy(dst_hbm.at[pl.ds(off, CH)], didx)
    pltpu.async_copy(y_hbm.at[sidx], rows, sem).wait()
    pltpu.sync_copy(rows, agg_sp.at[didx], add=True)
    return carry

  lax.fori_loop(0, NCHUNK, step, 0)
  plsc.subcore_barrier()
  _copy_out(c, s, row0, agg_sp, agg_out)


_sc_segsum = pl.kernel(
    _segsum_body,
    out_type=[jax.ShapeDtypeStruct((NC, N, F), jnp.float32)],
    mesh=_mesh,
    scratch_types=[
        pltpu.VMEM((CH,), jnp.int32),
        pltpu.VMEM((CH,), jnp.int32),
        pltpu.VMEM((CH, F), jnp.float32),
        pltpu.VMEM((SR, F), jnp.float32),
        pltpu.VMEM_SHARED((N, F), jnp.float32),
        pltpu.SemaphoreType.DMA,
    ],
    name="sc_segsum",
)


def _deg_body(dst_hbm, deg_out, didx, ones, zbuf, deg_sp):
  c = lax.axis_index("c")
  s = lax.axis_index("s")
  tid = c * NS + s
  row0 = _zero_shared(s, zbuf, deg_sp)
  ov = jnp.full((16,), 1.0, jnp.float32)

  def ofill(r, carry):
    for cc in range(F // 16):
      ones[r, pl.ds(cc * 16, 16)] = ov
    return carry

  lax.fori_loop(0, CH, ofill, 0)
  plsc.subcore_barrier()

  ebase = tid * EPT

  def step(k, carry):
    off = pl.multiple_of(ebase + k * CH, 8)
    pltpu.sync_copy(dst_hbm.at[pl.ds(off, CH)], didx)
    pltpu.sync_copy(ones, deg_sp.at[didx], add=True)
    return carry

  lax.fori_loop(0, NCHUNK, step, 0)
  plsc.subcore_barrier()
  _copy_out(c, s, row0, deg_sp, deg_out)


_sc_deg = pl.kernel(
    _deg_body,
    out_type=[jax.ShapeDtypeStruct((NC, N, F), jnp.float32)],
    mesh=_mesh,
    scratch_types=[
        pltpu.VMEM((CH,), jnp.int32),
        pltpu.VMEM((CH, F), jnp.float32),
        pltpu.VMEM((SR, F), jnp.float32),
        pltpu.VMEM_SHARED((N, F), jnp.float32),
    ],
    name="sc_deg",
)

BN = 400


def _ln_relu(h, g, b):
  m = jnp.mean(h, axis=-1, keepdims=True)
  v = jnp.mean((h - m) * (h - m), axis=-1, keepdims=True)
  h = (h - m) * lax.rsqrt(v + 1e-5) * g + b
  return jnp.maximum(h, 0.0)


def _sage_block(aggp, degp, x, wlT, bl, wrT, g, be):
  deg = degp[0, :, 0:1] + degp[1, :, 0:1]
  agg = (aggp[0] + aggp[1]) / jnp.maximum(deg, 1.0)
  h = (jnp.dot(agg, wlT[...], preferred_element_type=jnp.float32) + bl[...]
       + jnp.dot(x[...], wrT[...], preferred_element_type=jnp.float32))
  return _ln_relu(h, g[...], be[...])


def _tc1_body(aggp, degp, x, wlT, bl, wrT, g, be, h_out):
  h_out[...] = _sage_block(aggp, degp, x, wlT, bl, wrT, g, be)


def _tc2_body(aggp, degp, x, wlT, bl, wrT, g, be, wpT, bp, out, acc):
  i = pl.program_id(0)

  @pl.when(i == 0)
  def _():
    acc[...] = jnp.zeros_like(acc)

  h = _sage_block(aggp, degp, x, wlT, bl, wrT, g, be)
  acc[...] += jnp.sum(h, axis=0, keepdims=True)

  @pl.when(i == pl.num_programs(0) - 1)
  def _():
    pooled = acc[...] * (1.0 / N)
    out[...] = (jnp.dot(pooled, wpT[...], preferred_element_type=jnp.float32)
                + bp[...])


_row_spec = pl.BlockSpec((BN, F), lambda i: (i, 0))
_part_spec = pl.BlockSpec((NC, BN, F), lambda i: (0, i, 0))
_w_spec = pl.BlockSpec((F, F), lambda i: (0, 0))
_v_spec = pl.BlockSpec((1, F), lambda i: (0, 0))

_tc1 = pl.pallas_call(
    _tc1_body,
    grid=(N // BN,),
    in_specs=[_part_spec, _part_spec, _row_spec,
              _w_spec, _v_spec, _w_spec, _v_spec, _v_spec],
    out_specs=_row_spec,
    out_shape=jax.ShapeDtypeStruct((N, F), jnp.float32),
)

_tc2 = pl.pallas_call(
    _tc2_body,
    grid=(N // BN,),
    in_specs=[_part_spec, _part_spec, _row_spec,
              _w_spec, _v_spec, _w_spec, _v_spec, _v_spec,
              _w_spec, _v_spec],
    out_specs=pl.BlockSpec((1, F), lambda i: (0, 0)),
    out_shape=jax.ShapeDtypeStruct((1, F), jnp.float32),
    scratch_shapes=[pltpu.VMEM((1, F), jnp.float32)],
)


def kernel(x, edge_index, batch, W_l1, b_l1, W_r1, g1, be1,
           W_l2, b_l2, W_r2, g2, be2, W_p, b_p):
  src = edge_index[0]
  dst = edge_index[1]
  (degp,) = _sc_deg(dst)
  (agg1p,) = _sc_segsum(x, src, dst)
  h1 = _tc1(agg1p, degp, x,
            W_l1.T, b_l1.reshape(1, F), W_r1.T, g1.reshape(1, F),
            be1.reshape(1, F))
  (agg2p,) = _sc_segsum(h1, src, dst)
  out = _tc2(agg2p, degp, h1,
             W_l2.T, b_l2.reshape(1, F), W_r2.T, g2.reshape(1, F),
             be2.reshape(1, F), W_p.T, b_p.reshape(1, F))
  return out.reshape(F)


# speedup vs baseline: 7.4516x; 1.5867x over previous
import functools

import jax
import jax.numpy as jnp
from jax import lax
from jax.experimental import pallas as pl
from jax.experimental.pallas import tpu as pltpu
from jax.experimental.pallas import tpu_sc as plsc

N = 10000
E = 320000
F = 128

NC, NS = 2, 16
NW = NC * NS
EPT = E // NW
CH = 80
NCHUNK = EPT // CH
RA = 624
SR = 48
NCOPY = RA // SR
TAIL0 = NS * RA
TAIL = N - TAIL0

_mesh = plsc.VectorSubcoreMesh(
    core_axis_name="c", subcore_axis_name="s", num_cores=NC, num_subcores=NS
)


def _zero_shared(s, zbuf, sp):
  zv = jnp.zeros((16,), jnp.float32)

  def zfill(r, carry):
    for cc in range(F // 16):
      zbuf[r, pl.ds(cc * 16, 16)] = zv
    return carry

  lax.fori_loop(0, SR, zfill, 0)
  row0 = pl.multiple_of(s * RA, 8)
  for b in range(NCOPY):
    pltpu.sync_copy(zbuf, sp.at[pl.ds(row0 + b * SR, SR)])

  @pl.when(s == NS - 1)
  def _():
    pltpu.sync_copy(zbuf.at[pl.ds(0, TAIL)], sp.at[pl.ds(TAIL0, TAIL)])

  return row0


def _copy_out(c, s, row0, sp, out):
  for b in range(NCOPY):
    r = row0 + b * SR
    pltpu.sync_copy(sp.at[pl.ds(r, SR)], out.at[c].at[pl.ds(r, SR)])

  @pl.when(s == NS - 1)
  def _():
    pltpu.sync_copy(sp.at[pl.ds(TAIL0, TAIL)], out.at[c].at[pl.ds(TAIL0, TAIL)])


def _segsum_body(y_hbm, src_hbm, dst_hbm, agg_out,
                 sidxA, didxA, rowsA, semA, sidxB, didxB, rowsB, semB,
                 zbuf, agg_sp):
  c = lax.axis_index("c")
  s = lax.axis_index("s")
  tid = c * NS + s
  row0 = _zero_shared(s, zbuf, agg_sp)
  plsc.subcore_barrier()

  ebase = tid * EPT

  def fetch(ck, sidx, didx):
    off = pl.multiple_of(ebase + ck * CH, 8)
    pltpu.sync_copy(src_hbm.at[pl.ds(off, CH)], sidx)
    pltpu.sync_copy(dst_hbm.at[pl.ds(off, CH)], didx)

  fetch(0, sidxA, didxA)
  pltpu.async_copy(y_hbm.at[sidxA], rowsA, semA)

  def pair(k2, carry):
    cb = 2 * k2 + 1
    fetch(cb, sidxB, didxB)
    pltpu.async_copy(y_hbm.at[sidxB], rowsB, semB)
    pltpu.make_async_copy(y_hbm.at[sidxA], rowsA, semA).wait()
    pltpu.sync_copy(rowsA, agg_sp.at[didxA], add=True)
    fetch(cb + 1, sidxA, didxA)
    pltpu.async_copy(y_hbm.at[sidxA], rowsA, semA)
    pltpu.make_async_copy(y_hbm.at[sidxB], rowsB, semB).wait()
    pltpu.sync_copy(rowsB, agg_sp.at[didxB], add=True)
    return carry

  lax.fori_loop(0, (NCHUNK - 1) // 2, pair, 0)
  pltpu.make_async_copy(y_hbm.at[sidxA], rowsA, semA).wait()
  pltpu.sync_copy(rowsA, agg_sp.at[didxA], add=True)
  plsc.subcore_barrier()
  _copy_out(c, s, row0, agg_sp, agg_out)


_sc_segsum = pl.kernel(
    _segsum_body,
    out_type=[jax.ShapeDtypeStruct((NC, N, F), jnp.float32)],
    mesh=_mesh,
    scratch_types=[
        pltpu.VMEM((CH,), jnp.int32),
        pltpu.VMEM((CH,), jnp.int32),
        pltpu.VMEM((CH, F), jnp.float32),
        pltpu.SemaphoreType.DMA,
        pltpu.VMEM((CH,), jnp.int32),
        pltpu.VMEM((CH,), jnp.int32),
        pltpu.VMEM((CH, F), jnp.float32),
        pltpu.SemaphoreType.DMA,
        pltpu.VMEM((SR, F), jnp.float32),
        pltpu.VMEM_SHARED((N, F), jnp.float32),
    ],
    name="sc_segsum",
)


def _deg_body(dst_hbm, deg_out, didxA, semA, didxB, semB, ones, zbuf, deg_sp):
  c = lax.axis_index("c")
  s = lax.axis_index("s")
  tid = c * NS + s
  row0 = _zero_shared(s, zbuf, deg_sp)
  ov = jnp.full((16,), 1.0, jnp.float32)

  def ofill(r, carry):
    for cc in range(F // 16):
      ones[r, pl.ds(cc * 16, 16)] = ov
    return carry

  lax.fori_loop(0, CH, ofill, 0)
  plsc.subcore_barrier()

  ebase = tid * EPT

  def issue(ck, didx, sem):
    off = pl.multiple_of(ebase + ck * CH, 8)
    return pltpu.async_copy(dst_hbm.at[pl.ds(off, CH)], didx, sem)

  def drain(ck, didx, sem):
    off = pl.multiple_of(ebase + ck * CH, 8)
    pltpu.make_async_copy(dst_hbm.at[pl.ds(off, CH)], didx, sem).wait()
    pltpu.sync_copy(ones, deg_sp.at[didx], add=True)

  issue(0, didxA, semA)

  def pair(k2, carry):
    cb = 2 * k2 + 1
    issue(cb, didxB, semB)
    drain(2 * k2, didxA, semA)
    issue(cb + 1, didxA, semA)
    drain(cb, didxB, semB)
    return carry

  lax.fori_loop(0, (NCHUNK - 1) // 2, pair, 0)
  drain(NCHUNK - 1, didxA, semA)
  plsc.subcore_barrier()
  _copy_out(c, s, row0, deg_sp, deg_out)


_sc_deg = pl.kernel(
    _deg_body,
    out_type=[jax.ShapeDtypeStruct((NC, N, F), jnp.float32)],
    mesh=_mesh,
    scratch_types=[
        pltpu.VMEM((CH,), jnp.int32),
        pltpu.SemaphoreType.DMA,
        pltpu.VMEM((CH,), jnp.int32),
        pltpu.SemaphoreType.DMA,
        pltpu.VMEM((CH, F), jnp.float32),
        pltpu.VMEM((SR, F), jnp.float32),
        pltpu.VMEM_SHARED((N, F), jnp.float32),
    ],
    name="sc_deg",
)

BN = 400


def _ln_relu(h, g, b):
  m = jnp.mean(h, axis=-1, keepdims=True)
  v = jnp.mean((h - m) * (h - m), axis=-1, keepdims=True)
  h = (h - m) * lax.rsqrt(v + 1e-5) * g + b
  return jnp.maximum(h, 0.0)


def _sage_block(aggp, degp, x, wlT, bl, wrT, g, be):
  deg = degp[0, :, 0:1] + degp[1, :, 0:1]
  agg = (aggp[0] + aggp[1]) / jnp.maximum(deg, 1.0)
  h = (jnp.dot(agg, wlT[...], preferred_element_type=jnp.float32) + bl[...]
       + jnp.dot(x[...], wrT[...], preferred_element_type=jnp.float32))
  return _ln_relu(h, g[...], be[...])


def _tc1_body(aggp, degp, x, wlT, bl, wrT, g, be, h_out):
  h_out[...] = _sage_block(aggp, degp, x, wlT, bl, wrT, g, be)


def _tc2_body(aggp, degp, x, wlT, bl, wrT, g, be, wpT, bp, out, acc):
  i = pl.program_id(0)

  @pl.when(i == 0)
  def _():
    acc[...] = jnp.zeros_like(acc)

  h = _sage_block(aggp, degp, x, wlT, bl, wrT, g, be)
  acc[...] += jnp.sum(h, axis=0, keepdims=True)

  @pl.when(i == pl.num_programs(0) - 1)
  def _():
    pooled = acc[...] * (1.0 / N)
    out[...] = (jnp.dot(pooled, wpT[...], preferred_element_type=jnp.float32)
                + bp[...])


_row_spec = pl.BlockSpec((BN, F), lambda i: (i, 0))
_part_spec = pl.BlockSpec((NC, BN, F), lambda i: (0, i, 0))
_w_spec = pl.BlockSpec((F, F), lambda i: (0, 0))
_v_spec = pl.BlockSpec((1, F), lambda i: (0, 0))

_tc1 = pl.pallas_call(
    _tc1_body,
    grid=(N // BN,),
    in_specs=[_part_spec, _part_spec, _row_spec,
              _w_spec, _v_spec, _w_spec, _v_spec, _v_spec],
    out_specs=_row_spec,
    out_shape=jax.ShapeDtypeStruct((N, F), jnp.float32),
)

_tc2 = pl.pallas_call(
    _tc2_body,
    grid=(N // BN,),
    in_specs=[_part_spec, _part_spec, _row_spec,
              _w_spec, _v_spec, _w_spec, _v_spec, _v_spec,
              _w_spec, _v_spec],
    out_specs=pl.BlockSpec((1, F), lambda i: (0, 0)),
    out_shape=jax.ShapeDtypeStruct((1, F), jnp.float32),
    scratch_shapes=[pltpu.VMEM((1, F), jnp.float32)],
)


def kernel(x, edge_index, batch, W_l1, b_l1, W_r1, g1, be1,
           W_l2, b_l2, W_r2, g2, be2, W_p, b_p):
  src = edge_index[0]
  dst = edge_index[1]
  (degp,) = _sc_deg(dst)
  (agg1p,) = _sc_segsum(x, src, dst)
  h1 = _tc1(agg1p, degp, x,
            W_l1.T, b_l1.reshape(1, F), W_r1.T, g1.reshape(1, F),
            be1.reshape(1, F))
  (agg2p,) = _sc_segsum(h1, src, dst)
  out = _tc2(agg2p, degp, h1,
             W_l2.T, b_l2.reshape(1, F), W_r2.T, g2.reshape(1, F),
             be2.reshape(1, F), W_p.T, b_p.reshape(1, F))
  return out.reshape(F)


# speedup vs baseline: 9.1536x; 1.2284x over previous
import functools

import jax
import jax.numpy as jnp
from jax import lax
from jax.experimental import pallas as pl
from jax.experimental.pallas import tpu as pltpu
from jax.experimental.pallas import tpu_sc as plsc

N = 10000
E = 320000
F = 128

NC, NS = 2, 16
NW = NC * NS
EPT = E // NW
CH = 80
NCHUNK = EPT // CH
RA = 624
SR = 48
NCOPY = RA // SR
TAIL0 = NS * RA
TAIL = N - TAIL0

_mesh = plsc.VectorSubcoreMesh(
    core_axis_name="c", subcore_axis_name="s", num_cores=NC, num_subcores=NS
)


def _zero_shared(s, zbuf, sp):
  zv = jnp.zeros((16,), jnp.float32)

  def zfill(r, carry):
    for cc in range(F // 16):
      zbuf[r, pl.ds(cc * 16, 16)] = zv
    return carry

  lax.fori_loop(0, SR, zfill, 0)
  row0 = pl.multiple_of(s * RA, 8)
  for b in range(NCOPY):
    pltpu.sync_copy(zbuf, sp.at[pl.ds(row0 + b * SR, SR)])

  @pl.when(s == NS - 1)
  def _():
    pltpu.sync_copy(zbuf.at[pl.ds(0, TAIL)], sp.at[pl.ds(TAIL0, TAIL)])

  return row0


def _copy_out(c, s, row0, sp, out):
  for b in range(NCOPY):
    r = row0 + b * SR
    pltpu.sync_copy(sp.at[pl.ds(r, SR)], out.at[c].at[pl.ds(r, SR)])

  @pl.when(s == NS - 1)
  def _():
    pltpu.sync_copy(sp.at[pl.ds(TAIL0, TAIL)], out.at[c].at[pl.ds(TAIL0, TAIL)])


NB = 4


def _segsum_body(y_hbm, src_hbm, dst_hbm, agg_out, *rest):
  sidx = rest[0:NB]
  didx = rest[NB:2 * NB]
  rows = rest[2 * NB:3 * NB]
  gsem = rest[3 * NB:4 * NB]
  ssem = rest[4 * NB:5 * NB]
  zbuf, agg_sp = rest[5 * NB:]
  c = lax.axis_index("c")
  s = lax.axis_index("s")
  tid = c * NS + s
  row0 = _zero_shared(s, zbuf, agg_sp)
  plsc.subcore_barrier()

  ebase = tid * EPT

  def fetch_and_gather(ck, j):
    off = pl.multiple_of(ebase + ck * CH, 8)
    pltpu.sync_copy(src_hbm.at[pl.ds(off, CH)], sidx[j])
    pltpu.sync_copy(dst_hbm.at[pl.ds(off, CH)], didx[j])
    pltpu.async_copy(y_hbm.at[sidx[j]], rows[j], gsem[j])

  def drain_gather_issue_scatter(j):
    pltpu.make_async_copy(y_hbm.at[sidx[j]], rows[j], gsem[j]).wait()
    pltpu.async_copy(rows[j], agg_sp.at[didx[j]], ssem[j], add=True)

  def wait_scatter(j):
    pltpu.make_async_copy(rows[j], agg_sp.at[didx[j]], ssem[j]).wait()

  fetch_and_gather(0, 0)
  fetch_and_gather(1, 1)
  fetch_and_gather(2, 2)
  drain_gather_issue_scatter(0)
  fetch_and_gather(3, 3)
  drain_gather_issue_scatter(1)

  def ring(k4, carry):
    t0 = 4 * k4 + 4
    for j in range(NB):
      wait_scatter(j)
      fetch_and_gather(t0 + j, j)
      drain_gather_issue_scatter((j + 2) % NB)
    return carry

  lax.fori_loop(0, (NCHUNK - 5) // 4, ring, 0)
  wait_scatter(0)
  fetch_and_gather(NCHUNK - 1, 0)
  drain_gather_issue_scatter(2)
  drain_gather_issue_scatter(3)
  drain_gather_issue_scatter(0)
  wait_scatter(1)
  wait_scatter(2)
  wait_scatter(3)
  wait_scatter(0)
  plsc.subcore_barrier()
  _copy_out(c, s, row0, agg_sp, agg_out)


_sc_segsum = pl.kernel(
    _segsum_body,
    out_type=[jax.ShapeDtypeStruct((NC, N, F), jnp.float32)],
    mesh=_mesh,
    scratch_types=(
        [pltpu.VMEM((CH,), jnp.int32) for _ in range(NB)]
        + [pltpu.VMEM((CH,), jnp.int32) for _ in range(NB)]
        + [pltpu.VMEM((CH, F), jnp.float32) for _ in range(NB)]
        + [pltpu.SemaphoreType.DMA for _ in range(NB)]
        + [pltpu.SemaphoreType.DMA for _ in range(NB)]
        + [pltpu.VMEM((SR, F), jnp.float32),
           pltpu.VMEM_SHARED((N, F), jnp.float32)]
    ),
    name="sc_segsum",
)


def _deg_body(dst_hbm, deg_out, didxA, semA, didxB, semB, ones, zbuf, deg_sp):
  c = lax.axis_index("c")
  s = lax.axis_index("s")
  tid = c * NS + s
  row0 = _zero_shared(s, zbuf, deg_sp)
  ov = jnp.full((16,), 1.0, jnp.float32)

  def ofill(r, carry):
    for cc in range(F // 16):
      ones[r, pl.ds(cc * 16, 16)] = ov
    return carry

  lax.fori_loop(0, CH, ofill, 0)
  plsc.subcore_barrier()

  ebase = tid * EPT

  def issue(ck, didx, sem):
    off = pl.multiple_of(ebase + ck * CH, 8)
    return pltpu.async_copy(dst_hbm.at[pl.ds(off, CH)], didx, sem)

  def drain(ck, didx, sem):
    off = pl.multiple_of(ebase + ck * CH, 8)
    pltpu.make_async_copy(dst_hbm.at[pl.ds(off, CH)], didx, sem).wait()
    pltpu.sync_copy(ones, deg_sp.at[didx], add=True)

  issue(0, didxA, semA)

  def pair(k2, carry):
    cb = 2 * k2 + 1
    issue(cb, didxB, semB)
    drain(2 * k2, didxA, semA)
    issue(cb + 1, didxA, semA)
    drain(cb, didxB, semB)
    return carry

  lax.fori_loop(0, (NCHUNK - 1) // 2, pair, 0)
  drain(NCHUNK - 1, didxA, semA)
  plsc.subcore_barrier()
  _copy_out(c, s, row0, deg_sp, deg_out)


_sc_deg = pl.kernel(
    _deg_body,
    out_type=[jax.ShapeDtypeStruct((NC, N, F), jnp.float32)],
    mesh=_mesh,
    scratch_types=[
        pltpu.VMEM((CH,), jnp.int32),
        pltpu.SemaphoreType.DMA,
        pltpu.VMEM((CH,), jnp.int32),
        pltpu.SemaphoreType.DMA,
        pltpu.VMEM((CH, F), jnp.float32),
        pltpu.VMEM((SR, F), jnp.float32),
        pltpu.VMEM_SHARED((N, F), jnp.float32),
    ],
    name="sc_deg",
)

BN = 400


def _ln_relu(h, g, b):
  m = jnp.mean(h, axis=-1, keepdims=True)
  v = jnp.mean((h - m) * (h - m), axis=-1, keepdims=True)
  h = (h - m) * lax.rsqrt(v + 1e-5) * g + b
  return jnp.maximum(h, 0.0)


def _sage_block(aggp, degp, x, wlT, bl, wrT, g, be):
  deg = degp[0, :, 0:1] + degp[1, :, 0:1]
  agg = (aggp[0] + aggp[1]) / jnp.maximum(deg, 1.0)
  h = (jnp.dot(agg, wlT[...], preferred_element_type=jnp.float32) + bl[...]
       + jnp.dot(x[...], wrT[...], preferred_element_type=jnp.float32))
  return _ln_relu(h, g[...], be[...])


def _tc1_body(aggp, degp, x, wlT, bl, wrT, g, be, h_out):
  h_out[...] = _sage_block(aggp, degp, x, wlT, bl, wrT, g, be)


def _tc2_body(aggp, degp, x, wlT, bl, wrT, g, be, wpT, bp, out, acc):
  i = pl.program_id(0)

  @pl.when(i == 0)
  def _():
    acc[...] = jnp.zeros_like(acc)

  h = _sage_block(aggp, degp, x, wlT, bl, wrT, g, be)
  acc[...] += jnp.sum(h, axis=0, keepdims=True)

  @pl.when(i == pl.num_programs(0) - 1)
  def _():
    pooled = acc[...] * (1.0 / N)
    out[...] = (jnp.dot(pooled, wpT[...], preferred_element_type=jnp.float32)
                + bp[...])


_row_spec = pl.BlockSpec((BN, F), lambda i: (i, 0))
_part_spec = pl.BlockSpec((NC, BN, F), lambda i: (0, i, 0))
_w_spec = pl.BlockSpec((F, F), lambda i: (0, 0))
_v_spec = pl.BlockSpec((1, F), lambda i: (0, 0))

_tc1 = pl.pallas_call(
    _tc1_body,
    grid=(N // BN,),
    in_specs=[_part_spec, _part_spec, _row_spec,
              _w_spec, _v_spec, _w_spec, _v_spec, _v_spec],
    out_specs=_row_spec,
    out_shape=jax.ShapeDtypeStruct((N, F), jnp.float32),
)

_tc2 = pl.pallas_call(
    _tc2_body,
    grid=(N // BN,),
    in_specs=[_part_spec, _part_spec, _row_spec,
              _w_spec, _v_spec, _w_spec, _v_spec, _v_spec,
              _w_spec, _v_spec],
    out_specs=pl.BlockSpec((1, F), lambda i: (0, 0)),
    out_shape=jax.ShapeDtypeStruct((1, F), jnp.float32),
    scratch_shapes=[pltpu.VMEM((1, F), jnp.float32)],
)


def kernel(x, edge_index, batch, W_l1, b_l1, W_r1, g1, be1,
           W_l2, b_l2, W_r2, g2, be2, W_p, b_p):
  src = edge_index[0]
  dst = edge_index[1]
  (degp,) = _sc_deg(dst)
  (agg1p,) = _sc_segsum(x, src, dst)
  h1 = _tc1(agg1p, degp, x,
            W_l1.T, b_l1.reshape(1, F), W_r1.T, g1.reshape(1, F),
            be1.reshape(1, F))
  (agg2p,) = _sc_segsum(h1, src, dst)
  out = _tc2(agg2p, degp, h1,
             W_l2.T, b_l2.reshape(1, F), W_r2.T, g2.reshape(1, F),
             be2.reshape(1, F), W_p.T, b_p.reshape(1, F))
  return out.reshape(F)


# speedup vs baseline: 9.4042x; 1.0274x over previous
import functools

import jax
import jax.numpy as jnp
from jax import lax
from jax.experimental import pallas as pl
from jax.experimental.pallas import tpu as pltpu
from jax.experimental.pallas import tpu_sc as plsc

N = 10000
E = 320000
F = 128

NC, NS = 2, 16
NW = NC * NS
EPT = E // NW
CH = 80
NCHUNK = EPT // CH
RA = 624
SR = 48
NCOPY = RA // SR
TAIL0 = NS * RA
TAIL = N - TAIL0

_mesh = plsc.VectorSubcoreMesh(
    core_axis_name="c", subcore_axis_name="s", num_cores=NC, num_subcores=NS
)


def _zero_shared(s, zbuf, sp, sem):
  zv = jnp.zeros((16,), jnp.float32)

  def zfill(r, carry):
    for cc in range(F // 16):
      zbuf[r, pl.ds(cc * 16, 16)] = zv
    return carry

  lax.fori_loop(0, SR, zfill, 0)
  row0 = pl.multiple_of(s * RA, 8)
  for b in range(NCOPY):
    pltpu.async_copy(zbuf, sp.at[pl.ds(row0 + b * SR, SR)], sem)

  @pl.when(s == NS - 1)
  def _():
    pltpu.sync_copy(zbuf.at[pl.ds(0, TAIL)], sp.at[pl.ds(TAIL0, TAIL)])

  for b in range(NCOPY):
    pltpu.make_async_copy(zbuf, sp.at[pl.ds(row0 + b * SR, SR)], sem).wait()
  return row0


def _copy_out(c, s, row0, sp, out, sem):
  for b in range(NCOPY):
    r = row0 + b * SR
    pltpu.async_copy(sp.at[pl.ds(r, SR)], out.at[c].at[pl.ds(r, SR)], sem)

  @pl.when(s == NS - 1)
  def _():
    pltpu.sync_copy(sp.at[pl.ds(TAIL0, TAIL)], out.at[c].at[pl.ds(TAIL0, TAIL)])

  for b in range(NCOPY):
    r = row0 + b * SR
    pltpu.make_async_copy(sp.at[pl.ds(r, SR)], out.at[c].at[pl.ds(r, SR)],
                          sem).wait()


NB = 4


def _segsum_body(y_hbm, src_hbm, dst_hbm, agg_out, *rest):
  sidx = rest[0:NB]
  didx = rest[NB:2 * NB]
  rows = rest[2 * NB:3 * NB]
  gsem = rest[3 * NB:4 * NB]
  ssem = rest[4 * NB:5 * NB]
  zbuf, agg_sp = rest[5 * NB:]
  c = lax.axis_index("c")
  s = lax.axis_index("s")
  tid = c * NS + s
  row0 = _zero_shared(s, zbuf, agg_sp, gsem[0])
  plsc.subcore_barrier()

  ebase = tid * EPT

  def fetch_and_gather(ck, j):
    off = pl.multiple_of(ebase + ck * CH, 8)
    pltpu.sync_copy(src_hbm.at[pl.ds(off, CH)], sidx[j])
    pltpu.sync_copy(dst_hbm.at[pl.ds(off, CH)], didx[j])
    pltpu.async_copy(y_hbm.at[sidx[j]], rows[j], gsem[j])

  def drain_gather_issue_scatter(j):
    pltpu.make_async_copy(y_hbm.at[sidx[j]], rows[j], gsem[j]).wait()
    pltpu.async_copy(rows[j], agg_sp.at[didx[j]], ssem[j], add=True)

  def wait_scatter(j):
    pltpu.make_async_copy(rows[j], agg_sp.at[didx[j]], ssem[j]).wait()

  fetch_and_gather(0, 0)
  fetch_and_gather(1, 1)
  fetch_and_gather(2, 2)
  drain_gather_issue_scatter(0)
  fetch_and_gather(3, 3)
  drain_gather_issue_scatter(1)

  def ring(k4, carry):
    t0 = 4 * k4 + 4
    for j in range(NB):
      wait_scatter(j)
      fetch_and_gather(t0 + j, j)
      drain_gather_issue_scatter((j + 2) % NB)
    return carry

  lax.fori_loop(0, (NCHUNK - 5) // 4, ring, 0)
  wait_scatter(0)
  fetch_and_gather(NCHUNK - 1, 0)
  drain_gather_issue_scatter(2)
  drain_gather_issue_scatter(3)
  drain_gather_issue_scatter(0)
  wait_scatter(1)
  wait_scatter(2)
  wait_scatter(3)
  wait_scatter(0)
  plsc.subcore_barrier()
  _copy_out(c, s, row0, agg_sp, agg_out, gsem[0])


_sc_segsum = pl.kernel(
    _segsum_body,
    out_type=[jax.ShapeDtypeStruct((NC, N, F), jnp.float32)],
    mesh=_mesh,
    scratch_types=(
        [pltpu.VMEM((CH,), jnp.int32) for _ in range(NB)]
        + [pltpu.VMEM((CH,), jnp.int32) for _ in range(NB)]
        + [pltpu.VMEM((CH, F), jnp.float32) for _ in range(NB)]
        + [pltpu.SemaphoreType.DMA for _ in range(NB)]
        + [pltpu.SemaphoreType.DMA for _ in range(NB)]
        + [pltpu.VMEM((SR, F), jnp.float32),
           pltpu.VMEM_SHARED((N, F), jnp.float32)]
    ),
    name="sc_segsum",
)


def _deg_body(dst_hbm, deg_out, didxA, semA, didxB, semB, ones, zbuf, deg_sp):
  c = lax.axis_index("c")
  s = lax.axis_index("s")
  tid = c * NS + s
  row0 = _zero_shared(s, zbuf, deg_sp, semA)
  ov = jnp.full((16,), 1.0, jnp.float32)

  def ofill(r, carry):
    for cc in range(F // 16):
      ones[r, pl.ds(cc * 16, 16)] = ov
    return carry

  lax.fori_loop(0, CH, ofill, 0)
  plsc.subcore_barrier()

  ebase = tid * EPT

  def issue(ck, didx, sem):
    off = pl.multiple_of(ebase + ck * CH, 8)
    return pltpu.async_copy(dst_hbm.at[pl.ds(off, CH)], didx, sem)

  def drain(ck, didx, sem):
    off = pl.multiple_of(ebase + ck * CH, 8)
    pltpu.make_async_copy(dst_hbm.at[pl.ds(off, CH)], didx, sem).wait()
    pltpu.sync_copy(ones, deg_sp.at[didx], add=True)

  issue(0, didxA, semA)

  def pair(k2, carry):
    cb = 2 * k2 + 1
    issue(cb, didxB, semB)
    drain(2 * k2, didxA, semA)
    issue(cb + 1, didxA, semA)
    drain(cb, didxB, semB)
    return carry

  lax.fori_loop(0, (NCHUNK - 1) // 2, pair, 0)
  drain(NCHUNK - 1, didxA, semA)
  plsc.subcore_barrier()
  _copy_out(c, s, row0, deg_sp, deg_out, semA)


_sc_deg = pl.kernel(
    _deg_body,
    out_type=[jax.ShapeDtypeStruct((NC, N, F), jnp.float32)],
    mesh=_mesh,
    scratch_types=[
        pltpu.VMEM((CH,), jnp.int32),
        pltpu.SemaphoreType.DMA,
        pltpu.VMEM((CH,), jnp.int32),
        pltpu.SemaphoreType.DMA,
        pltpu.VMEM((CH, F), jnp.float32),
        pltpu.VMEM((SR, F), jnp.float32),
        pltpu.VMEM_SHARED((N, F), jnp.float32),
    ],
    name="sc_deg",
)

BN = 400


def _ln_relu(h, g, b):
  m = jnp.mean(h, axis=-1, keepdims=True)
  v = jnp.mean((h - m) * (h - m), axis=-1, keepdims=True)
  h = (h - m) * lax.rsqrt(v + 1e-5) * g + b
  return jnp.maximum(h, 0.0)


def _sage_block(aggp, degp, x, wlT, bl, wrT, g, be):
  deg = degp[0, :, 0:1] + degp[1, :, 0:1]
  agg = (aggp[0] + aggp[1]) / jnp.maximum(deg, 1.0)
  h = (jnp.dot(agg, wlT[...], preferred_element_type=jnp.float32) + bl[...]
       + jnp.dot(x[...], wrT[...], preferred_element_type=jnp.float32))
  return _ln_relu(h, g[...], be[...])


def _tc1_body(aggp, degp, x, wlT, bl, wrT, g, be, h_out):
  h_out[...] = _sage_block(aggp, degp, x, wlT, bl, wrT, g, be)


def _tc2_body(aggp, degp, x, wlT, bl, wrT, g, be, wpT, bp, out, acc):
  i = pl.program_id(0)

  @pl.when(i == 0)
  def _():
    acc[...] = jnp.zeros_like(acc)

  h = _sage_block(aggp, degp, x, wlT, bl, wrT, g, be)
  acc[...] += jnp.sum(h, axis=0, keepdims=True)

  @pl.when(i == pl.num_programs(0) - 1)
  def _():
    pooled = acc[...] * (1.0 / N)
    out[...] = (jnp.dot(pooled, wpT[...], preferred_element_type=jnp.float32)
                + bp[...])


_row_spec = pl.BlockSpec((BN, F), lambda i: (i, 0))
_part_spec = pl.BlockSpec((NC, BN, F), lambda i: (0, i, 0))
_w_spec = pl.BlockSpec((F, F), lambda i: (0, 0))
_v_spec = pl.BlockSpec((1, F), lambda i: (0, 0))

_tc1 = pl.pallas_call(
    _tc1_body,
    grid=(N // BN,),
    in_specs=[_part_spec, _part_spec, _row_spec,
              _w_spec, _v_spec, _w_spec, _v_spec, _v_spec],
    out_specs=_row_spec,
    out_shape=jax.ShapeDtypeStruct((N, F), jnp.float32),
)

_tc2 = pl.pallas_call(
    _tc2_body,
    grid=(N // BN,),
    in_specs=[_part_spec, _part_spec, _row_spec,
              _w_spec, _v_spec, _w_spec, _v_spec, _v_spec,
              _w_spec, _v_spec],
    out_specs=pl.BlockSpec((1, F), lambda i: (0, 0)),
    out_shape=jax.ShapeDtypeStruct((1, F), jnp.float32),
    scratch_shapes=[pltpu.VMEM((1, F), jnp.float32)],
)


def kernel(x, edge_index, batch, W_l1, b_l1, W_r1, g1, be1,
           W_l2, b_l2, W_r2, g2, be2, W_p, b_p):
  src = edge_index[0]
  dst = edge_index[1]
  (degp,) = _sc_deg(dst)
  (agg1p,) = _sc_segsum(x, src, dst)
  h1 = _tc1(agg1p, degp, x,
            W_l1.T, b_l1.reshape(1, F), W_r1.T, g1.reshape(1, F),
            be1.reshape(1, F))
  (agg2p,) = _sc_segsum(h1, src, dst)
  out = _tc2(agg2p, degp, h1,
             W_l2.T, b_l2.reshape(1, F), W_r2.T, g2.reshape(1, F),
             be2.reshape(1, F), W_p.T, b_p.reshape(1, F))
  return out.reshape(F)


# speedup vs baseline: 11.0855x; 1.1788x over previous
import functools

import jax
import jax.numpy as jnp
from jax import lax
from jax.experimental import pallas as pl
from jax.experimental.pallas import tpu as pltpu
from jax.experimental.pallas import tpu_sc as plsc

N = 10000
E = 320000
F = 128

NC, NS = 2, 16
NW = NC * NS
EPT = E // NW
CH = 80
NCHUNK = EPT // CH
RA = 624
SR = 48
NCOPY = RA // SR
TAIL0 = NS * RA
TAIL = N - TAIL0

_mesh = plsc.VectorSubcoreMesh(
    core_axis_name="c", subcore_axis_name="s", num_cores=NC, num_subcores=NS
)


def _zero_shared(s, zbuf, sp, sem):
  zv = jnp.zeros((16,), jnp.float32)

  def zfill(r, carry):
    for cc in range(F // 16):
      zbuf[r, pl.ds(cc * 16, 16)] = zv
    return carry

  lax.fori_loop(0, SR, zfill, 0)
  row0 = pl.multiple_of(s * RA, 8)
  for b in range(NCOPY):
    pltpu.async_copy(zbuf, sp.at[pl.ds(row0 + b * SR, SR)], sem)

  @pl.when(s == NS - 1)
  def _():
    pltpu.sync_copy(zbuf.at[pl.ds(0, TAIL)], sp.at[pl.ds(TAIL0, TAIL)])

  for b in range(NCOPY):
    pltpu.make_async_copy(zbuf, sp.at[pl.ds(row0 + b * SR, SR)], sem).wait()
  return row0


def _copy_out(c, s, row0, sp, out, sem):
  for b in range(NCOPY):
    r = row0 + b * SR
    pltpu.async_copy(sp.at[pl.ds(r, SR)], out.at[c].at[pl.ds(r, SR)], sem)

  @pl.when(s == NS - 1)
  def _():
    pltpu.sync_copy(sp.at[pl.ds(TAIL0, TAIL)], out.at[c].at[pl.ds(TAIL0, TAIL)])

  for b in range(NCOPY):
    r = row0 + b * SR
    pltpu.make_async_copy(sp.at[pl.ds(r, SR)], out.at[c].at[pl.ds(r, SR)],
                          sem).wait()


NB = 4
NI = 8
NCHT = E // CH


def _segsum_body(y_hbm, epk_hbm, agg_out, *rest):
  ibuf = rest[0:NI]
  isem = rest[NI:2 * NI]
  rows = rest[2 * NI:2 * NI + NB]
  gsem = rest[2 * NI + NB:2 * NI + 2 * NB]
  ssem = rest[2 * NI + 2 * NB:2 * NI + 3 * NB]
  zbuf, agg_sp = rest[2 * NI + 3 * NB:]
  c = lax.axis_index("c")
  s = lax.axis_index("s")
  tid = c * NS + s
  row0 = _zero_shared(s, zbuf, agg_sp, gsem[0])
  plsc.subcore_barrier()

  cbase = tid * NCHUNK

  def F(ck, j8):
    pltpu.async_copy(epk_hbm.at[cbase + ck], ibuf[j8], isem[j8])

  def WI(ck, j8):
    pltpu.make_async_copy(epk_hbm.at[cbase + ck], ibuf[j8], isem[j8]).wait()

  def G(ck, j8, j4):
    pltpu.async_copy(y_hbm.at[ibuf[j8].at[0]], rows[j4], gsem[j4])

  def D(ck, j8, j4):
    pltpu.make_async_copy(y_hbm.at[ibuf[j8].at[0]], rows[j4],
                          gsem[j4]).wait()
    pltpu.async_copy(rows[j4], agg_sp.at[ibuf[j8].at[1]], ssem[j4], add=True)

  def WS(ck, j8, j4):
    pltpu.make_async_copy(rows[j4], agg_sp.at[ibuf[j8].at[1]],
                          ssem[j4]).wait()

  F(0, 0)
  F(1, 1)
  for t in range(0, 4):
    F(t + 2, (t + 2) % NI)
    WI(t, t % NI)
    G(t, t % NI, t % NB)
    if t >= 2:
      D(t - 2, (t - 2) % NI, (t - 2) % NB)

  def ring(k8, carry):
    t0 = 8 * k8 + 4
    for i in range(8):
      t = t0 + i
      WS(t - 4, i % NI, i % NB)
      F(t + 2, (i + 6) % NI, )
      WI(t, (i + 4) % NI)
      G(t, (i + 4) % NI, i % NB)
      D(t - 2, (i + 2) % NI, (i + 2) % NB)
    return carry

  lax.fori_loop(0, 14, ring, 0)
  for t in range(116, NCHUNK):
    WS(t - 4, (t - 4) % NI, (t - 4) % NB)
    if t + 2 < NCHUNK:
      F(t + 2, (t + 2) % NI)
    WI(t, t % NI)
    G(t, t % NI, t % NB)
    D(t - 2, (t - 2) % NI, (t - 2) % NB)
  D(NCHUNK - 2, (NCHUNK - 2) % NI, (NCHUNK - 2) % NB)
  D(NCHUNK - 1, (NCHUNK - 1) % NI, (NCHUNK - 1) % NB)
  for ck in range(NCHUNK - 4, NCHUNK):
    WS(ck, ck % NI, ck % NB)
  plsc.subcore_barrier()
  _copy_out(c, s, row0, agg_sp, agg_out, gsem[0])


_sc_segsum = pl.kernel(
    _segsum_body,
    out_type=[jax.ShapeDtypeStruct((NC, N, F), jnp.float32)],
    mesh=_mesh,
    scratch_types=(
        [pltpu.VMEM((2, CH), jnp.int32) for _ in range(NI)]
        + [pltpu.SemaphoreType.DMA for _ in range(NI)]
        + [pltpu.VMEM((CH, F), jnp.float32) for _ in range(NB)]
        + [pltpu.SemaphoreType.DMA for _ in range(NB)]
        + [pltpu.SemaphoreType.DMA for _ in range(NB)]
        + [pltpu.VMEM((SR, F), jnp.float32),
           pltpu.VMEM_SHARED((N, F), jnp.float32)]
    ),
    name="sc_segsum",
)


def _deg_body(dst_hbm, deg_out, didxA, semA, didxB, semB, ones, zbuf, deg_sp):
  c = lax.axis_index("c")
  s = lax.axis_index("s")
  tid = c * NS + s
  row0 = _zero_shared(s, zbuf, deg_sp, semA)
  ov = jnp.full((16,), 1.0, jnp.float32)

  def ofill(r, carry):
    for cc in range(F // 16):
      ones[r, pl.ds(cc * 16, 16)] = ov
    return carry

  lax.fori_loop(0, CH, ofill, 0)
  plsc.subcore_barrier()

  ebase = tid * EPT

  def issue(ck, didx, sem):
    off = pl.multiple_of(ebase + ck * CH, 8)
    return pltpu.async_copy(dst_hbm.at[pl.ds(off, CH)], didx, sem)

  def drain(ck, didx, sem):
    off = pl.multiple_of(ebase + ck * CH, 8)
    pltpu.make_async_copy(dst_hbm.at[pl.ds(off, CH)], didx, sem).wait()
    pltpu.sync_copy(ones, deg_sp.at[didx], add=True)

  issue(0, didxA, semA)

  def pair(k2, carry):
    cb = 2 * k2 + 1
    issue(cb, didxB, semB)
    drain(2 * k2, didxA, semA)
    issue(cb + 1, didxA, semA)
    drain(cb, didxB, semB)
    return carry

  lax.fori_loop(0, (NCHUNK - 1) // 2, pair, 0)
  drain(NCHUNK - 1, didxA, semA)
  plsc.subcore_barrier()
  _copy_out(c, s, row0, deg_sp, deg_out, semA)


_sc_deg = pl.kernel(
    _deg_body,
    out_type=[jax.ShapeDtypeStruct((NC, N, F), jnp.float32)],
    mesh=_mesh,
    scratch_types=[
        pltpu.VMEM((CH,), jnp.int32),
        pltpu.SemaphoreType.DMA,
        pltpu.VMEM((CH,), jnp.int32),
        pltpu.SemaphoreType.DMA,
        pltpu.VMEM((CH, F), jnp.float32),
        pltpu.VMEM((SR, F), jnp.float32),
        pltpu.VMEM_SHARED((N, F), jnp.float32),
    ],
    name="sc_deg",
)

BN = 400


def _ln_relu(h, g, b):
  m = jnp.mean(h, axis=-1, keepdims=True)
  v = jnp.mean((h - m) * (h - m), axis=-1, keepdims=True)
  h = (h - m) * lax.rsqrt(v + 1e-5) * g + b
  return jnp.maximum(h, 0.0)


def _sage_block(aggp, degp, x, wlT, bl, wrT, g, be):
  deg = degp[0, :, 0:1] + degp[1, :, 0:1]
  agg = (aggp[0] + aggp[1]) / jnp.maximum(deg, 1.0)
  h = (jnp.dot(agg, wlT[...], preferred_element_type=jnp.float32) + bl[...]
       + jnp.dot(x[...], wrT[...], preferred_element_type=jnp.float32))
  return _ln_relu(h, g[...], be[...])


def _tc1_body(aggp, degp, x, wlT, bl, wrT, g, be, h_out):
  h_out[...] = _sage_block(aggp, degp, x, wlT, bl, wrT, g, be)


def _tc2_body(aggp, degp, x, wlT, bl, wrT, g, be, wpT, bp, out, acc):
  i = pl.program_id(0)

  @pl.when(i == 0)
  def _():
    acc[...] = jnp.zeros_like(acc)

  h = _sage_block(aggp, degp, x, wlT, bl, wrT, g, be)
  acc[...] += jnp.sum(h, axis=0, keepdims=True)

  @pl.when(i == pl.num_programs(0) - 1)
  def _():
    pooled = acc[...] * (1.0 / N)
    out[...] = (jnp.dot(pooled, wpT[...], preferred_element_type=jnp.float32)
                + bp[...])


_row_spec = pl.BlockSpec((BN, F), lambda i: (i, 0))
_part_spec = pl.BlockSpec((NC, BN, F), lambda i: (0, i, 0))
_w_spec = pl.BlockSpec((F, F), lambda i: (0, 0))
_v_spec = pl.BlockSpec((1, F), lambda i: (0, 0))

_tc1 = pl.pallas_call(
    _tc1_body,
    grid=(N // BN,),
    in_specs=[_part_spec, _part_spec, _row_spec,
              _w_spec, _v_spec, _w_spec, _v_spec, _v_spec],
    out_specs=_row_spec,
    out_shape=jax.ShapeDtypeStruct((N, F), jnp.float32),
)

_tc2 = pl.pallas_call(
    _tc2_body,
    grid=(N // BN,),
    in_specs=[_part_spec, _part_spec, _row_spec,
              _w_spec, _v_spec, _w_spec, _v_spec, _v_spec,
              _w_spec, _v_spec],
    out_specs=pl.BlockSpec((1, F), lambda i: (0, 0)),
    out_shape=jax.ShapeDtypeStruct((1, F), jnp.float32),
    scratch_shapes=[pltpu.VMEM((1, F), jnp.float32)],
)


def kernel(x, edge_index, batch, W_l1, b_l1, W_r1, g1, be1,
           W_l2, b_l2, W_r2, g2, be2, W_p, b_p):
  dst = edge_index[1]
  epk = edge_index.reshape(2, NCHT, CH).transpose(1, 0, 2)
  (degp,) = _sc_deg(dst)
  (agg1p,) = _sc_segsum(x, epk)
  h1 = _tc1(agg1p, degp, x,
            W_l1.T, b_l1.reshape(1, F), W_r1.T, g1.reshape(1, F),
            be1.reshape(1, F))
  (agg2p,) = _sc_segsum(h1, epk)
  out = _tc2(agg2p, degp, h1,
             W_l2.T, b_l2.reshape(1, F), W_r2.T, g2.reshape(1, F),
             be2.reshape(1, F), W_p.T, b_p.reshape(1, F))
  return out.reshape(F)
